# baseline (speedup 1.0000x reference)
import jax
import jax.numpy as jnp
from jax import lax
from jax.experimental import pallas as pl

FEATS_ = 128
K_ = 50000
HALF_ = K_ // 2
BLOCK_ = 5000


def _scale2_kernel(x1_ref, x2_ref, w_ref, o_ref):
    w = w_ref[...]
    inv_norm = jax.lax.rsqrt(jnp.sum(w * w))
    x1 = x1_ref[...]
    s1 = jnp.dot(x1, w, preferred_element_type=jnp.float32) * inv_norm
    o_ref[0] = x1 * jnp.tanh(s1)
    x2 = x2_ref[...]
    s2 = jnp.dot(x2, w, preferred_element_type=jnp.float32) * inv_norm
    o_ref[1] = x2 * jnp.tanh(s2)


def kernel(node_embs, mask, scorer):
    del mask
    n_blocks = HALF_ // BLOCK_
    out = pl.pallas_call(
        _scale2_kernel,
        grid=(n_blocks,),
        in_specs=[
            pl.BlockSpec((BLOCK_, FEATS_), lambda i: (i, 0)),
            pl.BlockSpec((BLOCK_, FEATS_), lambda i: (i + HALF_ // BLOCK_, 0)),
            pl.BlockSpec((FEATS_, 1), lambda i: (0, 0)),
        ],
        out_specs=pl.BlockSpec((2, BLOCK_, FEATS_), lambda i: (0, i, 0)),
        out_shape=jax.ShapeDtypeStruct((2, HALF_, FEATS_), jnp.float32),
    )(node_embs, node_embs, scorer)
    return out.reshape(K_, FEATS_).T


# speedup vs baseline: 1.0045x; 1.0045x over previous
import jax
import jax.numpy as jnp
from jax.experimental import pallas as pl
from jax.experimental.pallas import tpu as pltpu

FEATS_ = 128
K_ = 50000
CHUNK_ = 2000
NCHUNK_ = K_ // CHUNK_
NBUF_ = 4


def _manual_kernel(x_hbm, w_ref, o_hbm, xbuf, ybuf, insem, outsem):
    w = w_ref[...]
    inv_norm = jax.lax.rsqrt(jnp.sum(w * w))

    def in_copy(j, slot):
        return pltpu.make_async_copy(
            x_hbm.at[pl.ds(j * CHUNK_, CHUNK_), :], xbuf.at[slot],
            insem.at[slot])

    def out_copy(j, slot):
        return pltpu.make_async_copy(
            ybuf.at[slot], o_hbm.at[pl.ds(j * CHUNK_, CHUNK_), :],
            outsem.at[slot])

    for j in range(min(NBUF_, NCHUNK_)):
        in_copy(j, j % NBUF_).start()

    for j in range(NCHUNK_):
        slot = j % NBUF_
        in_copy(j, slot).wait()
        if j >= NBUF_:
            out_copy(j - NBUF_, slot).wait()
        x = xbuf[slot]
        s = jnp.dot(x, w, preferred_element_type=jnp.float32) * inv_norm
        ybuf[slot] = x * jnp.tanh(s)
        out_copy(j, slot).start()
        if j + NBUF_ < NCHUNK_:
            in_copy(j + NBUF_, slot).start()

    for j in range(max(NCHUNK_ - NBUF_, 0), NCHUNK_):
        out_copy(j, j % NBUF_).wait()


def kernel(node_embs, mask, scorer):
    del mask
    out = pl.pallas_call(
        _manual_kernel,
        in_specs=[
            pl.BlockSpec(memory_space=pl.ANY),
            pl.BlockSpec(memory_space=pltpu.VMEM),
        ],
        out_specs=pl.BlockSpec(memory_space=pl.ANY),
        out_shape=jax.ShapeDtypeStruct((K_, FEATS_), jnp.float32),
        scratch_shapes=[
            pltpu.VMEM((NBUF_, CHUNK_, FEATS_), jnp.float32),
            pltpu.VMEM((NBUF_, CHUNK_, FEATS_), jnp.float32),
            pltpu.SemaphoreType.DMA((NBUF_,)),
            pltpu.SemaphoreType.DMA((NBUF_,)),
        ],
    )(node_embs, scorer)
    return out.T
